# baseline (speedup 1.0000x reference)
import functools

import jax
import jax.numpy as jnp
from jax import lax
from jax.experimental import pallas as pl

_IOU_THRESHOLD = 0.7
_B = 8
_T = 24


def _sig(x):
    return 1.0 / (1.0 + jnp.exp(-x))


def _scale_body(gx, gy, off, nbo, pred_ref, gtv_ref, ancw_r, anch_r,
                ancw_c, anch_c, out_ref):
    i = pl.program_id(0)
    S = gx * gy
    nb = 3 * S
    f32 = jnp.float32

    def rows(c):
        return jnp.concatenate(
            [pred_ref[0, 85 * a + c:85 * a + c + 1, :] for a in range(3)],
            axis=0)

    xr, yr, wr, hr, objr = rows(0), rows(1), rows(2), rows(3), rows(4)
    lane = lax.broadcasted_iota(jnp.int32, (3, S), 1)
    xi = (lane % gx).astype(f32)
    yi = (lane // gx).astype(f32)
    aw = ancw_c[off:off + 3, 0:1]
    ah = anch_c[off:off + 3, 0:1]
    px = _sig(xr) + xi / gx
    py = _sig(yr) + yi / gy
    pw = aw * jnp.exp(wr)
    ph = ah * jnp.exp(hr)
    bx1 = px - pw / 2
    bx2 = px + pw / 2
    by1 = py - ph / 2
    by2 = py + ph / 2
    ab = (bx2 - bx1) * (by2 - by1)
    obj = _sig(objr)
    bf16 = jnp.bfloat16
    nidx = (lax.broadcasted_iota(jnp.int32, (3, S), 0) * S
            + lane).astype(jnp.int16)
    _R = _IOU_THRESHOLD / (1.0 + _IOU_THRESHOLD)
    qb = (_R * (ab + 1e-9)).astype(bf16)
    bx1b = bx1.astype(bf16)
    bx2b = bx2.astype(bf16)
    by1b = by1.astype(bf16)
    by2b = by2.astype(bf16)

    gv = gtv_ref[0]
    g0 = gv[:, 0:1]
    g1 = gv[:, 1:2]
    g2 = gv[:, 2:3]
    g3 = gv[:, 3:4]
    g4 = gv[:, 4:5]
    cellx = jnp.clip(jnp.floor(g0 * gx), 0, gx - 1)
    celly = jnp.clip(jnp.floor(g1 * gy), 0, gy - 1)
    dx = g0 - cellx / gx
    dy = g1 - celly / gy
    bv = jnp.full((_T, 1), -1.0, f32)
    bj = jnp.zeros((_T, 1), jnp.int32)
    for j in range(9):
        awj = ancw_r[0:1, j:j + 1]
        ahj = anch_r[0:1, j:j + 1]
        inter = jnp.minimum(g2, awj) * jnp.minimum(g3, ahj)
        union = g2 * g3 + awj * ahj - inter
        r = inter / (union + 1e-9)
        bj = jnp.where(r > bv, j, bj)
        bv = jnp.maximum(r, bv)
    maskv = (bj >= off) & (bj <= off + 2)
    bbf = (bj - off).astype(f32) * S + celly * gx + cellx
    bbi = bbf.astype(jnp.int32)
    cx = dx + cellx / gx
    cy = dy + celly / gy
    gx1 = cx - g2 / 2
    gx2 = cx + g2 / 2
    gy1 = cy - g3 / 2
    gy2 = cy + g3 / 2
    aa = (gx2 - gx1) * (gy2 - gy1)
    sa = (_R * aa).astype(bf16)
    gx1b = gx1.astype(bf16)
    gx2b = gx2.astype(bf16)
    gy1b = gy1.astype(bf16)
    gy2b = gy2.astype(bf16)
    bbiM = jnp.where(maskv, bbi, -1).astype(jnp.int16)

    hit = jnp.zeros((3, S), jnp.bool_)
    occ = jnp.zeros((3, S), jnp.bool_)
    zb = jnp.zeros((), bf16)
    for t in range(_T):
        ltx = jnp.maximum(gx1b[t:t + 1, 0:1], bx1b)
        lty = jnp.maximum(gy1b[t:t + 1, 0:1], by1b)
        rbx = jnp.minimum(gx2b[t:t + 1, 0:1], bx2b)
        rby = jnp.minimum(gy2b[t:t + 1, 0:1], by2b)
        inter = jnp.maximum(rbx - ltx, zb) * jnp.maximum(rby - lty, zb)
        hit = hit | (inter > qb + sa[t:t + 1, 0:1])
        occ = occ | (nidx == bbiM[t:t + 1, 0:1])
    neg = (~hit) & (~occ)
    noobj = jnp.sum(jnp.where(neg, obj * obj, 0.0))

    pos = (celly * gx + cellx).astype(jnp.int32)
    at = jnp.clip(bj - off, 0, 2)
    H = (lax.broadcasted_iota(jnp.int32, (_T, S), 1) == pos).astype(f32)
    Pv = pred_ref[0]
    G = lax.dot_general(H, Pv, (((1,), (1,)), ((), ())),
                        preferred_element_type=f32)
    pb = jnp.where(at == 0, G[:, 0:85],
                   jnp.where(at == 1, G[:, 85:170], G[:, 170:255]))
    ci = lax.broadcasted_iota(jnp.int32, (_T, 85), 1)
    pbs = jnp.where((ci == 2) | (ci == 3), pb, _sig(pb))
    awsel = jnp.zeros((_T, 1), f32)
    ahsel = jnp.zeros((_T, 1), f32)
    for j in range(9):
        awsel = jnp.where(bj == j, ancw_r[0:1, j:j + 1], awsel)
        ahsel = jnp.where(bj == j, anch_r[0:1, j:j + 1], ahsel)
    t2 = jnp.log(g2) - jnp.log(awsel)
    t3 = jnp.log(g3) - jnp.log(ahsel)
    gidx_f = bbf + (i * nb + nbo).astype(f32)
    vf = (maskv & (gidx_f != 0.0)).astype(f32)
    sw = 2.0 - g2 * g3
    oh = (ci - 5 == g4.astype(jnp.int32)).astype(f32)
    tgt = jnp.where(ci == 0, dx,
          jnp.where(ci == 1, dy,
          jnp.where(ci == 2, t2,
          jnp.where(ci == 3, t3,
          jnp.where(ci == 4, 1.0, oh)))))
    d2 = (pbs - tgt) ** 2
    coord = jnp.sum(jnp.where(ci < 4, vf * sw * d2, 0.0))
    objl = jnp.sum(jnp.where(ci == 4, vf * d2, 0.0))
    clsl = jnp.sum(jnp.where(ci >= 5, vf * d2, 0.0))

    i4 = lax.broadcasted_iota(jnp.int32, (1, 4), 1)
    part = jnp.where(i4 == 0, noobj,
           jnp.where(i4 == 1, coord,
           jnp.where(i4 == 2, objl, clsl)))

    @pl.when(i == 0)
    def _():
        out_ref[0:1, :] = part

    @pl.when(i != 0)
    def _():
        out_ref[0:1, :] = out_ref[0:1, :] + part


def _scale_call(pred, gt, ancw_r, anch_r, ancw_c, anch_c, gx, gy, off, nbo):
    S = gx * gy
    body = functools.partial(_scale_body, gx, gy, off, nbo)
    return pl.pallas_call(
        body,
        grid=(_B,),
        in_specs=[
            pl.BlockSpec((1, 255, S), lambda i: (i, 0, 0)),
            pl.BlockSpec((1, _T, 5), lambda i: (i, 0, 0)),
            pl.BlockSpec((1, 9), lambda i: (0, 0)),
            pl.BlockSpec((1, 9), lambda i: (0, 0)),
            pl.BlockSpec((9, 1), lambda i: (0, 0)),
            pl.BlockSpec((9, 1), lambda i: (0, 0)),
        ],
        out_specs=pl.BlockSpec((1, 4), lambda i: (0, 0)),
        out_shape=jax.ShapeDtypeStruct((1, 4), jnp.float32),
    )(pred, gt, ancw_r, anch_r, ancw_c, anch_c)


@jax.jit
def kernel(pred0, pred1, pred2, gt_boxes, anchors):
    anc = anchors[0, :, :, 0, 0]
    ancw_r = anc[:, 0].reshape(1, 9)
    anch_r = anc[:, 1].reshape(1, 9)
    ancw_c = anc[:, 0].reshape(9, 1)
    anch_c = anc[:, 1].reshape(9, 1)
    total = jnp.float32(0.0)
    nbo = 0
    for s, pred in enumerate((pred0, pred1, pred2)):
        B, C, gy, gx = pred.shape
        predr = pred.reshape(B, C, gy * gx)
        parts = _scale_call(predr, gt_boxes, ancw_r, anch_r, ancw_c, anch_c,
                            gx, gy, 3 * s, nbo)
        total = total + (parts[0, 3] + parts[0, 0] + parts[0, 2]
                         + 0.75 * parts[0, 1])
        nbo += B * 3 * gy * gx
    return total


# speedup vs baseline: 1.0256x; 1.0256x over previous
import functools

import jax
import jax.numpy as jnp
from jax import lax
from jax.experimental import pallas as pl

_IOU_THRESHOLD = 0.7
_B = 8
_T = 24


def _sig(x):
    return 1.0 / (1.0 + jnp.exp(-x))


def _scale_body(gx, gy, off, nbo, pred_ref, gtv_ref, ancw_r, anch_r,
                ancw_c, anch_c, out_ref):
    i = pl.program_id(0)
    S = gx * gy
    nb = 3 * S
    f32 = jnp.float32

    def rows(c):
        return jnp.concatenate(
            [pred_ref[0, 85 * a + c:85 * a + c + 1, :] for a in range(3)],
            axis=0)

    xr, yr, wr, hr, objr = rows(0), rows(1), rows(2), rows(3), rows(4)
    lane = lax.broadcasted_iota(jnp.int32, (3, S), 1)
    xi = (lane % gx).astype(f32)
    yi = (lane // gx).astype(f32)
    aw = ancw_c[off:off + 3, 0:1]
    ah = anch_c[off:off + 3, 0:1]
    px = _sig(xr) + xi / gx
    py = _sig(yr) + yi / gy
    pw = aw * jnp.exp(wr)
    ph = ah * jnp.exp(hr)
    bx1 = px - pw / 2
    bx2 = px + pw / 2
    by1 = py - ph / 2
    by2 = py + ph / 2
    ab = (bx2 - bx1) * (by2 - by1)
    obj = _sig(objr)
    nidx = lax.broadcasted_iota(jnp.int32, (3, S), 0) * S + lane
    _R = _IOU_THRESHOLD / (1.0 + _IOU_THRESHOLD)
    qb = _R * (ab + 1e-9)

    gv = gtv_ref[0]
    g0 = gv[:, 0:1]
    g1 = gv[:, 1:2]
    g2 = gv[:, 2:3]
    g3 = gv[:, 3:4]
    g4 = gv[:, 4:5]
    cellx = jnp.clip(jnp.floor(g0 * gx), 0, gx - 1)
    celly = jnp.clip(jnp.floor(g1 * gy), 0, gy - 1)
    dx = g0 - cellx / gx
    dy = g1 - celly / gy
    bv = jnp.full((_T, 1), -1.0, f32)
    bj = jnp.zeros((_T, 1), jnp.int32)
    for j in range(9):
        awj = ancw_r[0:1, j:j + 1]
        ahj = anch_r[0:1, j:j + 1]
        inter = jnp.minimum(g2, awj) * jnp.minimum(g3, ahj)
        union = g2 * g3 + awj * ahj - inter
        r = inter / (union + 1e-9)
        bj = jnp.where(r > bv, j, bj)
        bv = jnp.maximum(r, bv)
    maskv = (bj >= off) & (bj <= off + 2)
    bbf = (bj - off).astype(f32) * S + celly * gx + cellx
    bbi = bbf.astype(jnp.int32)
    cx = dx + cellx / gx
    cy = dy + celly / gy
    gx1 = cx - g2 / 2
    gx2 = cx + g2 / 2
    gy1 = cy - g3 / 2
    gy2 = cy + g3 / 2
    aa = (gx2 - gx1) * (gy2 - gy1)
    sa = _R * aa
    bbiM = jnp.where(maskv, bbi, -1)

    hit = jnp.zeros((3, S), jnp.bool_)
    occ = jnp.zeros((3, S), jnp.bool_)
    for t in range(_T):
        ltx = jnp.maximum(gx1[t:t + 1, 0:1], bx1)
        lty = jnp.maximum(gy1[t:t + 1, 0:1], by1)
        rbx = jnp.minimum(gx2[t:t + 1, 0:1], bx2)
        rby = jnp.minimum(gy2[t:t + 1, 0:1], by2)
        inter = jnp.maximum(rbx - ltx, 0.0) * jnp.maximum(rby - lty, 0.0)
        hit = hit | (inter > qb + sa[t:t + 1, 0:1])
        occ = occ | (nidx == bbiM[t:t + 1, 0:1])
    neg = (~hit) & (~occ)
    noobj = jnp.sum(jnp.where(neg, obj * obj, 0.0))

    pos = (celly * gx + cellx).astype(jnp.int32)
    at = jnp.clip(bj - off, 0, 2)
    H = (lax.broadcasted_iota(jnp.int32, (_T, S), 1) == pos).astype(f32)
    Pv = pred_ref[0]
    G = lax.dot_general(H, Pv, (((1,), (1,)), ((), ())),
                        preferred_element_type=f32)
    pb = jnp.where(at == 0, G[:, 0:85],
                   jnp.where(at == 1, G[:, 85:170], G[:, 170:255]))
    ci = lax.broadcasted_iota(jnp.int32, (_T, 85), 1)
    pbs = jnp.where((ci == 2) | (ci == 3), pb, _sig(pb))
    awsel = jnp.zeros((_T, 1), f32)
    ahsel = jnp.zeros((_T, 1), f32)
    for j in range(9):
        awsel = jnp.where(bj == j, ancw_r[0:1, j:j + 1], awsel)
        ahsel = jnp.where(bj == j, anch_r[0:1, j:j + 1], ahsel)
    t2 = jnp.log(g2) - jnp.log(awsel)
    t3 = jnp.log(g3) - jnp.log(ahsel)
    gidx_f = bbf + (i * nb + nbo).astype(f32)
    vf = (maskv & (gidx_f != 0.0)).astype(f32)
    sw = 2.0 - g2 * g3
    oh = (ci - 5 == g4.astype(jnp.int32)).astype(f32)
    tgt = jnp.where(ci == 0, dx,
          jnp.where(ci == 1, dy,
          jnp.where(ci == 2, t2,
          jnp.where(ci == 3, t3,
          jnp.where(ci == 4, 1.0, oh)))))
    d2 = (pbs - tgt) ** 2
    coord = jnp.sum(jnp.where(ci < 4, vf * sw * d2, 0.0))
    objl = jnp.sum(jnp.where(ci == 4, vf * d2, 0.0))
    clsl = jnp.sum(jnp.where(ci >= 5, vf * d2, 0.0))

    i4 = lax.broadcasted_iota(jnp.int32, (1, 4), 1)
    part = jnp.where(i4 == 0, noobj,
           jnp.where(i4 == 1, coord,
           jnp.where(i4 == 2, objl, clsl)))

    @pl.when(i == 0)
    def _():
        out_ref[0:1, :] = part

    @pl.when(i != 0)
    def _():
        out_ref[0:1, :] = out_ref[0:1, :] + part


def _scale_call(pred, gt, ancw_r, anch_r, ancw_c, anch_c, gx, gy, off, nbo):
    S = gx * gy
    body = functools.partial(_scale_body, gx, gy, off, nbo)
    return pl.pallas_call(
        body,
        grid=(_B,),
        in_specs=[
            pl.BlockSpec((1, 255, S), lambda i: (i, 0, 0)),
            pl.BlockSpec((1, _T, 5), lambda i: (i, 0, 0)),
            pl.BlockSpec((1, 9), lambda i: (0, 0)),
            pl.BlockSpec((1, 9), lambda i: (0, 0)),
            pl.BlockSpec((9, 1), lambda i: (0, 0)),
            pl.BlockSpec((9, 1), lambda i: (0, 0)),
        ],
        out_specs=pl.BlockSpec((1, 4), lambda i: (0, 0)),
        out_shape=jax.ShapeDtypeStruct((1, 4), jnp.float32),
    )(pred, gt, ancw_r, anch_r, ancw_c, anch_c)


@jax.jit
def kernel(pred0, pred1, pred2, gt_boxes, anchors):
    anc = anchors[0, :, :, 0, 0]
    ancw_r = anc[:, 0].reshape(1, 9)
    anch_r = anc[:, 1].reshape(1, 9)
    ancw_c = anc[:, 0].reshape(9, 1)
    anch_c = anc[:, 1].reshape(9, 1)
    total = jnp.float32(0.0)
    nbo = 0
    for s, pred in enumerate((pred0, pred1, pred2)):
        B, C, gy, gx = pred.shape
        predr = pred.reshape(B, C, gy * gx)
        parts = _scale_call(predr, gt_boxes, ancw_r, anch_r, ancw_c, anch_c,
                            gx, gy, 3 * s, nbo)
        total = total + (parts[0, 3] + parts[0, 0] + parts[0, 2]
                         + 0.75 * parts[0, 1])
        nbo += B * 3 * gy * gx
    return total
